# Initial kernel scaffold; baseline (speedup 1.0000x reference)
import jax
import jax.numpy as jnp
from jax.experimental import pallas as pl


def kernel(node_hidden, edge_index, edge_hidden, node_id, edge_id, W1, b1, W2, b2, ln_gamma, ln_beta):
    raise NotImplementedError("write your pallas kernel here")


# speedup vs baseline: 3.9067x; 3.9067x over previous
import functools

import jax
import jax.numpy as jnp
from jax import lax
from jax.experimental import pallas as pl
from jax.experimental.pallas import tpu as pltpu
from jax.experimental.pallas import tpu_sc as plsc

N = 10000
E = 320000
D = 128
NG = 512

NC = 2
NS = 16
NW = NC * NS

K = 128
NCHUNK = E // K
BASE_CH = NCHUNK // NW
EXTRA = NCHUNK % NW

NZ_FULL = N // K
NZ_TAIL = N - NZ_FULL * K

BN = 400
NB = N // BN


def _sc_aggregate_body(nh_hbm, esrc_hbm, edst_hbm, eh_hbm, out_hbm,
                       sidx_v, didx_v, rows_v, erows_v, aggr_sh):
    cid = lax.axis_index("c")
    sid = lax.axis_index("s")
    wid = sid * NC + cid

    def _zrow(i, carry):
        for j in range(D // 16):
            erows_v[i, pl.ds(j * 16, 16)] = jnp.zeros((16,), jnp.float32)
        return carry
    lax.fori_loop(0, K, _zrow, 0)

    for m in range(NZ_FULL):
        @pl.when(sid == (m % NS))
        def _():
            pltpu.sync_copy(erows_v, aggr_sh.at[pl.ds(m * K, K)])
    @pl.when(sid == (NZ_FULL % NS))
    def _():
        pltpu.sync_copy(erows_v.at[pl.ds(0, NZ_TAIL)],
                        aggr_sh.at[pl.ds(NZ_FULL * K, NZ_TAIL)])
    plsc.subcore_barrier()

    nch = BASE_CH + jnp.where(wid < EXTRA, 1, 0)

    def _chunk(k, carry):
        g = wid + NW * k
        base = g * K
        pltpu.sync_copy(esrc_hbm.at[pl.ds(base, K)], sidx_v)
        pltpu.sync_copy(edst_hbm.at[pl.ds(base, K)], didx_v)
        pltpu.sync_copy(nh_hbm.at[sidx_v], rows_v)
        pltpu.sync_copy(eh_hbm.at[pl.ds(base, K)], erows_v)

        def _crow(i, c):
            for j in range(D // 16):
                s = pl.ds(j * 16, 16)
                rows_v[i, s] = jnp.maximum(rows_v[i, s] + erows_v[i, s], 0.0)
            return c
        lax.fori_loop(0, K, _crow, 0)

        pltpu.sync_copy(rows_v, aggr_sh.at[didx_v], add=True)
        return carry

    lax.fori_loop(0, nch, _chunk, 0)
    plsc.subcore_barrier()

    for m in range(NZ_FULL):
        @pl.when(sid == (m % NS))
        def _():
            pltpu.sync_copy(aggr_sh.at[pl.ds(m * K, K)],
                            out_hbm.at[cid, pl.ds(m * K, K)])
    @pl.when(sid == (NZ_FULL % NS))
    def _():
        pltpu.sync_copy(aggr_sh.at[pl.ds(NZ_FULL * K, NZ_TAIL)],
                        out_hbm.at[cid, pl.ds(NZ_FULL * K, NZ_TAIL)])


_sc_aggregate = functools.partial(
    pl.kernel,
    out_type=jax.ShapeDtypeStruct((NC, N, D), jnp.float32),
    mesh=plsc.VectorSubcoreMesh(core_axis_name="c", subcore_axis_name="s"),
    scratch_types=[
        pltpu.VMEM((K,), jnp.int32),
        pltpu.VMEM((K,), jnp.int32),
        pltpu.VMEM((K, D), jnp.float32),
        pltpu.VMEM((K, D), jnp.float32),
        pltpu.VMEM_SHARED((N, D), jnp.float32),
    ],
)(_sc_aggregate_body)


def _counts_body(nid_ref, out_ref):
    i = pl.program_id(0)
    nid = nid_ref[0, 0, :]
    oh = (nid[:, None] == lax.broadcasted_iota(jnp.int32, (BN, NG), 1))
    colsum = jnp.sum(oh.astype(jnp.float32), axis=0)

    @pl.when(i == 0)
    def _():
        out_ref[...] = colsum[None, :]

    @pl.when(i > 0)
    def _():
        out_ref[...] = out_ref[...] + colsum[None, :]


def _dense_body(x_ref, p_ref, nid_ref, cnt_ref, w1_ref, b1_ref, w2_ref,
                b2_ref, g_ref, be_ref, out_ref):
    x = x_ref[...]
    h = x + p_ref[0] + p_ref[1]
    h = jnp.dot(h, w1_ref[...], preferred_element_type=jnp.float32,
                precision=lax.Precision.HIGHEST) + b1_ref[...]
    h = jnp.maximum(h, 0.0)
    h = jnp.dot(h, w2_ref[...], preferred_element_type=jnp.float32,
                precision=lax.Precision.HIGHEST) + b2_ref[...]
    mu = jnp.mean(h, axis=1, keepdims=True)
    xc = h - mu
    var = jnp.mean(xc * xc, axis=1, keepdims=True)
    h = xc * lax.rsqrt(var + 1e-5) * g_ref[...] + be_ref[...]
    nid = nid_ref[0, 0, :]
    rc = lax.rsqrt(jnp.maximum(cnt_ref[...], 1.0))
    oh = (nid[:, None] == lax.broadcasted_iota(jnp.int32, (BN, NG), 1))
    rinv = jnp.sum(oh.astype(jnp.float32) * rc, axis=1, keepdims=True)
    h = jnp.maximum(h * rinv, 0.0)
    out_ref[...] = h + x


def kernel(node_hidden, edge_index, edge_hidden, node_id, edge_id,
           W1, b1, W2, b2, ln_gamma, ln_beta):
    esrc = edge_index[0]
    edst = edge_index[1]
    partials = _sc_aggregate(node_hidden, esrc, edst, edge_hidden)

    nid3 = jnp.reshape(node_id.astype(jnp.int32), (NB, 1, BN))
    counts = pl.pallas_call(
        _counts_body,
        grid=(NB,),
        in_specs=[pl.BlockSpec((1, 1, BN), lambda i: (i, 0, 0))],
        out_specs=pl.BlockSpec((1, NG), lambda i: (0, 0)),
        out_shape=jax.ShapeDtypeStruct((1, NG), jnp.float32),
    )(nid3)

    out = pl.pallas_call(
        _dense_body,
        grid=(NB,),
        in_specs=[
            pl.BlockSpec((BN, D), lambda i: (i, 0)),
            pl.BlockSpec((NC, BN, D), lambda i: (0, i, 0)),
            pl.BlockSpec((1, 1, BN), lambda i: (i, 0, 0)),
            pl.BlockSpec((1, NG), lambda i: (0, 0)),
            pl.BlockSpec((D, 2 * D), lambda i: (0, 0)),
            pl.BlockSpec((1, 2 * D), lambda i: (0, 0)),
            pl.BlockSpec((2 * D, D), lambda i: (0, 0)),
            pl.BlockSpec((1, D), lambda i: (0, 0)),
            pl.BlockSpec((1, D), lambda i: (0, 0)),
            pl.BlockSpec((1, D), lambda i: (0, 0)),
        ],
        out_specs=pl.BlockSpec((BN, D), lambda i: (i, 0)),
        out_shape=jax.ShapeDtypeStruct((N, D), jnp.float32),
    )(node_hidden, partials, nid3, counts,
      W1, jnp.reshape(b1, (1, 2 * D)), W2, jnp.reshape(b2, (1, D)),
      jnp.reshape(ln_gamma, (1, D)), jnp.reshape(ln_beta, (1, D)))
    return out


# speedup vs baseline: 7.8381x; 2.0063x over previous
import functools

import jax
import jax.numpy as jnp
from jax import lax
from jax.experimental import pallas as pl
from jax.experimental.pallas import tpu as pltpu
from jax.experimental.pallas import tpu_sc as plsc

N = 10000
E = 320000
D = 128
NG = 512

NC = 2
NS = 16
NW = NC * NS

K = 80
CPW = E // K // NW

NZ = N // K

BN = 400
NB = N // BN


def _sc_aggregate_body(nh_hbm, es_hbm, ed_hbm, eh_hbm, out_hbm,
                       sx0, sx1, sx2, sx3, dx0, dx1, dx2, dx3,
                       rows0, rows1, er0, er1,
                       aggr_sh, si0, si1, si2, si3, sg0, sg1, ss0, ss1):
    cid = lax.axis_index("c")
    sid = lax.axis_index("s")
    wid = sid * NC + cid

    sidx = (sx0, sx1, sx2, sx3)
    didx = (dx0, dx1, dx2, dx3)
    rows = (rows0, rows1)
    er = (er0, er1)
    semi = (si0, si1, si2, si3)
    semg = (sg0, sg1)
    sems = (ss0, ss1)

    def base_of(c):
        return (wid * CPW + c) * K

    def issue_idx(c, s):
        pltpu.async_copy(es_hbm.at[pl.ds(base_of(c), K)], sidx[s], semi[s])
        pltpu.async_copy(ed_hbm.at[pl.ds(base_of(c), K)], didx[s], semi[s])

    def wait_idx(c, s):
        pltpu.make_async_copy(es_hbm.at[pl.ds(base_of(c), K)], sidx[s],
                              semi[s]).wait()
        pltpu.make_async_copy(ed_hbm.at[pl.ds(base_of(c), K)], didx[s],
                              semi[s]).wait()

    def issue_ge(c, b, s):
        pltpu.async_copy(nh_hbm.at[sidx[s]], rows[b], semg[b])
        pltpu.async_copy(eh_hbm.at[pl.ds(base_of(c), K)], er[b], semg[b])

    def wait_ge(c, b, s):
        pltpu.make_async_copy(nh_hbm.at[sidx[s]], rows[b], semg[b]).wait()
        pltpu.make_async_copy(eh_hbm.at[pl.ds(base_of(c), K)], er[b],
                              semg[b]).wait()

    def issue_sc(b, s):
        pltpu.async_copy(rows[b], aggr_sh.at[didx[s]], sems[b], add=True)

    def wait_sc(b, s):
        pltpu.make_async_copy(rows[b], aggr_sh.at[didx[s]], sems[b]).wait()

    def compute(b):
        @plsc.parallel_loop(0, K, unroll=2)
        def _(i):
            for j in range(D // 16):
                sl = pl.ds(j * 16, 16)
                rows[b][i, sl] = jnp.maximum(rows[b][i, sl] + er[b][i, sl],
                                             0.0)

    def _zrow(i, carry):
        for j in range(D // 16):
            er0[i, pl.ds(j * 16, 16)] = jnp.zeros((16,), jnp.float32)
        return carry
    lax.fori_loop(0, K, _zrow, 0)
    for m in range(NZ):
        @pl.when(sid == (m % NS))
        def _():
            pltpu.sync_copy(er0, aggr_sh.at[pl.ds(m * K, K)])
    plsc.subcore_barrier()

    issue_idx(0, 0)
    issue_idx(1, 1)
    wait_idx(0, 0)
    issue_ge(0, 0, 0)

    def group(c2, carry):
        for bb in range(4):
            c = c2 * 4 + bb
            b = bb % 2
            ob = 1 - b

            @pl.when(c >= 1)
            def _():
                wait_sc(ob, (bb + 3) % 4)

            @pl.when(c + 2 < CPW)
            def _():
                issue_idx(c + 2, (bb + 2) % 4)

            wait_idx(c + 1, (bb + 1) % 4)
            issue_ge(c + 1, ob, (bb + 1) % 4)
            wait_ge(c, b, bb)
            compute(b)
            issue_sc(b, bb)
        return carry

    lax.fori_loop(0, (CPW - 1) // 4, group, 0)

    wait_sc(1, 3)
    wait_ge(CPW - 1, 0, 0)
    compute(0)
    issue_sc(0, 0)
    wait_sc(0, 0)

    plsc.subcore_barrier()

    for m in range(NZ):
        @pl.when(sid == (m % NS))
        def _():
            pltpu.sync_copy(aggr_sh.at[pl.ds(m * K, K)],
                            out_hbm.at[cid, pl.ds(m * K, K)])


_sc_aggregate = functools.partial(
    pl.kernel,
    out_type=jax.ShapeDtypeStruct((NC, N, D), jnp.float32),
    mesh=plsc.VectorSubcoreMesh(core_axis_name="c", subcore_axis_name="s"),
    scratch_types=[
        pltpu.VMEM((K,), jnp.int32),
        pltpu.VMEM((K,), jnp.int32),
        pltpu.VMEM((K,), jnp.int32),
        pltpu.VMEM((K,), jnp.int32),
        pltpu.VMEM((K,), jnp.int32),
        pltpu.VMEM((K,), jnp.int32),
        pltpu.VMEM((K,), jnp.int32),
        pltpu.VMEM((K,), jnp.int32),
        pltpu.VMEM((K, D), jnp.float32),
        pltpu.VMEM((K, D), jnp.float32),
        pltpu.VMEM((K, D), jnp.float32),
        pltpu.VMEM((K, D), jnp.float32),
        pltpu.VMEM_SHARED((N, D), jnp.float32),
        pltpu.SemaphoreType.DMA,
        pltpu.SemaphoreType.DMA,
        pltpu.SemaphoreType.DMA,
        pltpu.SemaphoreType.DMA,
        pltpu.SemaphoreType.DMA,
        pltpu.SemaphoreType.DMA,
        pltpu.SemaphoreType.DMA,
        pltpu.SemaphoreType.DMA,
    ],
)(_sc_aggregate_body)


def _counts_body(nid_ref, out_ref):
    i = pl.program_id(0)
    nid = nid_ref[0, 0, :]
    oh = (nid[:, None] == lax.broadcasted_iota(jnp.int32, (BN, NG), 1))
    colsum = jnp.sum(oh.astype(jnp.float32), axis=0)

    @pl.when(i == 0)
    def _():
        out_ref[...] = colsum[None, :]

    @pl.when(i > 0)
    def _():
        out_ref[...] = out_ref[...] + colsum[None, :]


def _dense_body(x_ref, p_ref, nid_ref, cnt_ref, w1_ref, b1_ref, w2_ref,
                b2_ref, g_ref, be_ref, out_ref):
    x = x_ref[...]
    h = x + p_ref[0] + p_ref[1]
    h = jnp.dot(h, w1_ref[...], preferred_element_type=jnp.float32,
                precision=lax.Precision.HIGHEST) + b1_ref[...]
    h = jnp.maximum(h, 0.0)
    h = jnp.dot(h, w2_ref[...], preferred_element_type=jnp.float32,
                precision=lax.Precision.HIGHEST) + b2_ref[...]
    mu = jnp.mean(h, axis=1, keepdims=True)
    xc = h - mu
    var = jnp.mean(xc * xc, axis=1, keepdims=True)
    h = xc * lax.rsqrt(var + 1e-5) * g_ref[...] + be_ref[...]
    nid = nid_ref[0, 0, :]
    rc = lax.rsqrt(jnp.maximum(cnt_ref[...], 1.0))
    oh = (nid[:, None] == lax.broadcasted_iota(jnp.int32, (BN, NG), 1))
    rinv = jnp.sum(oh.astype(jnp.float32) * rc, axis=1, keepdims=True)
    h = jnp.maximum(h * rinv, 0.0)
    out_ref[...] = h + x


def kernel(node_hidden, edge_index, edge_hidden, node_id, edge_id,
           W1, b1, W2, b2, ln_gamma, ln_beta):
    partials = _sc_aggregate(node_hidden, edge_index[0], edge_index[1],
                             edge_hidden)

    nid3 = jnp.reshape(node_id.astype(jnp.int32), (NB, 1, BN))
    counts = pl.pallas_call(
        _counts_body,
        grid=(NB,),
        in_specs=[pl.BlockSpec((1, 1, BN), lambda i: (i, 0, 0))],
        out_specs=pl.BlockSpec((1, NG), lambda i: (0, 0)),
        out_shape=jax.ShapeDtypeStruct((1, NG), jnp.float32),
    )(nid3)

    out = pl.pallas_call(
        _dense_body,
        grid=(NB,),
        in_specs=[
            pl.BlockSpec((BN, D), lambda i: (i, 0)),
            pl.BlockSpec((NC, BN, D), lambda i: (0, i, 0)),
            pl.BlockSpec((1, 1, BN), lambda i: (i, 0, 0)),
            pl.BlockSpec((1, NG), lambda i: (0, 0)),
            pl.BlockSpec((D, 2 * D), lambda i: (0, 0)),
            pl.BlockSpec((1, 2 * D), lambda i: (0, 0)),
            pl.BlockSpec((2 * D, D), lambda i: (0, 0)),
            pl.BlockSpec((1, D), lambda i: (0, 0)),
            pl.BlockSpec((1, D), lambda i: (0, 0)),
            pl.BlockSpec((1, D), lambda i: (0, 0)),
        ],
        out_specs=pl.BlockSpec((BN, D), lambda i: (i, 0)),
        out_shape=jax.ShapeDtypeStruct((N, D), jnp.float32),
    )(node_hidden, partials, nid3, counts,
      W1, jnp.reshape(b1, (1, 2 * D)), W2, jnp.reshape(b2, (1, D)),
      jnp.reshape(ln_gamma, (1, D)), jnp.reshape(ln_beta, (1, D)))
    return out


# speedup vs baseline: 8.2553x; 1.0532x over previous
import functools

import jax
import jax.numpy as jnp
from jax import lax
from jax.experimental import pallas as pl
from jax.experimental.pallas import tpu as pltpu
from jax.experimental.pallas import tpu_sc as plsc

N = 10000
E = 320000
D = 128
NG = 512

NC = 2
NS = 16
NW = NC * NS

K = 80
CPW = E // K // NW

NZ = N // K

BN = 1000
NB = N // BN


def _sc_aggregate_body(nh_hbm, es_hbm, ed_hbm, eh_hbm, out_hbm,
                       sx0, sx1, sx2, sx3, dx0, dx1, dx2, dx3,
                       rows0, rows1, er0, er1,
                       aggr_sh, si0, si1, si2, si3, sg0, sg1, ss0, ss1):
    cid = lax.axis_index("c")
    sid = lax.axis_index("s")
    wid = sid * NC + cid

    sidx = (sx0, sx1, sx2, sx3)
    didx = (dx0, dx1, dx2, dx3)
    rows = (rows0, rows1)
    er = (er0, er1)
    semi = (si0, si1, si2, si3)
    semg = (sg0, sg1)
    sems = (ss0, ss1)

    def base_of(c):
        return (wid * CPW + c) * K

    def issue_idx(c, s):
        pltpu.async_copy(es_hbm.at[pl.ds(base_of(c), K)], sidx[s], semi[s])
        pltpu.async_copy(ed_hbm.at[pl.ds(base_of(c), K)], didx[s], semi[s])

    def wait_idx(c, s):
        pltpu.make_async_copy(es_hbm.at[pl.ds(base_of(c), K)], sidx[s],
                              semi[s]).wait()
        pltpu.make_async_copy(ed_hbm.at[pl.ds(base_of(c), K)], didx[s],
                              semi[s]).wait()

    def issue_ge(c, b, s):
        pltpu.async_copy(nh_hbm.at[sidx[s]], rows[b], semg[b])
        pltpu.async_copy(eh_hbm.at[pl.ds(base_of(c), K)], er[b], semg[b])

    def wait_ge(c, b, s):
        pltpu.make_async_copy(nh_hbm.at[sidx[s]], rows[b], semg[b]).wait()
        pltpu.make_async_copy(eh_hbm.at[pl.ds(base_of(c), K)], er[b],
                              semg[b]).wait()

    def issue_sc(b, s):
        pltpu.async_copy(rows[b], aggr_sh.at[didx[s]], sems[b], add=True)

    def wait_sc(b, s):
        pltpu.make_async_copy(rows[b], aggr_sh.at[didx[s]], sems[b]).wait()

    def compute(b):
        @plsc.parallel_loop(0, K, unroll=2)
        def _(i):
            for j in range(D // 16):
                sl = pl.ds(j * 16, 16)
                rows[b][i, sl] = jnp.maximum(rows[b][i, sl] + er[b][i, sl],
                                             0.0)

    def _zrow(i, carry):
        for j in range(D // 16):
            er0[i, pl.ds(j * 16, 16)] = jnp.zeros((16,), jnp.float32)
        return carry
    lax.fori_loop(0, K, _zrow, 0)
    for m in range(NZ):
        @pl.when(sid == (m % NS))
        def _():
            pltpu.sync_copy(er0, aggr_sh.at[pl.ds(m * K, K)])
    plsc.subcore_barrier()

    issue_idx(0, 0)
    issue_idx(1, 1)
    wait_idx(0, 0)
    issue_ge(0, 0, 0)

    def group(c2, carry):
        for bb in range(4):
            c = c2 * 4 + bb
            b = bb % 2
            ob = 1 - b

            @pl.when(c >= 1)
            def _():
                wait_sc(ob, (bb + 3) % 4)

            @pl.when(c + 2 < CPW)
            def _():
                issue_idx(c + 2, (bb + 2) % 4)

            wait_idx(c + 1, (bb + 1) % 4)
            issue_ge(c + 1, ob, (bb + 1) % 4)
            wait_ge(c, b, bb)
            compute(b)
            issue_sc(b, bb)
        return carry

    lax.fori_loop(0, (CPW - 1) // 4, group, 0)

    wait_sc(1, 3)
    wait_ge(CPW - 1, 0, 0)
    compute(0)
    issue_sc(0, 0)
    wait_sc(0, 0)

    plsc.subcore_barrier()

    for m in range(NZ):
        @pl.when(sid == (m % NS))
        def _():
            pltpu.sync_copy(aggr_sh.at[pl.ds(m * K, K)],
                            out_hbm.at[cid, pl.ds(m * K, K)])


_sc_aggregate = functools.partial(
    pl.kernel,
    out_type=jax.ShapeDtypeStruct((NC, N, D), jnp.float32),
    mesh=plsc.VectorSubcoreMesh(core_axis_name="c", subcore_axis_name="s"),
    scratch_types=[
        pltpu.VMEM((K,), jnp.int32),
        pltpu.VMEM((K,), jnp.int32),
        pltpu.VMEM((K,), jnp.int32),
        pltpu.VMEM((K,), jnp.int32),
        pltpu.VMEM((K,), jnp.int32),
        pltpu.VMEM((K,), jnp.int32),
        pltpu.VMEM((K,), jnp.int32),
        pltpu.VMEM((K,), jnp.int32),
        pltpu.VMEM((K, D), jnp.float32),
        pltpu.VMEM((K, D), jnp.float32),
        pltpu.VMEM((K, D), jnp.float32),
        pltpu.VMEM((K, D), jnp.float32),
        pltpu.VMEM_SHARED((N, D), jnp.float32),
        pltpu.SemaphoreType.DMA,
        pltpu.SemaphoreType.DMA,
        pltpu.SemaphoreType.DMA,
        pltpu.SemaphoreType.DMA,
        pltpu.SemaphoreType.DMA,
        pltpu.SemaphoreType.DMA,
        pltpu.SemaphoreType.DMA,
        pltpu.SemaphoreType.DMA,
    ],
)(_sc_aggregate_body)


def _counts_body(nid_ref, out_ref):
    i = pl.program_id(0)
    nid = nid_ref[0, 0, :]
    oh = (nid[:, None] == lax.broadcasted_iota(jnp.int32, (BN, NG), 1))
    colsum = jnp.sum(oh.astype(jnp.float32), axis=0)

    @pl.when(i == 0)
    def _():
        out_ref[...] = colsum[None, :]

    @pl.when(i > 0)
    def _():
        out_ref[...] = out_ref[...] + colsum[None, :]


def _dense_body(x_ref, p_ref, nid_ref, cnt_ref, w1_ref, b1_ref, w2_ref,
                b2_ref, g_ref, be_ref, out_ref):
    x = x_ref[...]
    h = x + p_ref[0] + p_ref[1]
    h = jnp.dot(h, w1_ref[...], preferred_element_type=jnp.float32) + b1_ref[...]
    h = jnp.maximum(h, 0.0)
    h = jnp.dot(h, w2_ref[...], preferred_element_type=jnp.float32) + b2_ref[...]
    mu = jnp.mean(h, axis=1, keepdims=True)
    xc = h - mu
    var = jnp.mean(xc * xc, axis=1, keepdims=True)
    h = xc * lax.rsqrt(var + 1e-5) * g_ref[...] + be_ref[...]
    nid = nid_ref[0, 0, :]
    rc = lax.rsqrt(jnp.maximum(cnt_ref[...], 1.0))
    oh = (nid[:, None] == lax.broadcasted_iota(jnp.int32, (BN, NG), 1))
    rinv = jnp.sum(oh.astype(jnp.float32) * rc, axis=1, keepdims=True)
    h = jnp.maximum(h * rinv, 0.0)
    out_ref[...] = h + x


def kernel(node_hidden, edge_index, edge_hidden, node_id, edge_id,
           W1, b1, W2, b2, ln_gamma, ln_beta):
    partials = _sc_aggregate(node_hidden, edge_index[0], edge_index[1],
                             edge_hidden)

    nid3 = jnp.reshape(node_id.astype(jnp.int32), (NB, 1, BN))
    counts = pl.pallas_call(
        _counts_body,
        grid=(NB,),
        in_specs=[pl.BlockSpec((1, 1, BN), lambda i: (i, 0, 0))],
        out_specs=pl.BlockSpec((1, NG), lambda i: (0, 0)),
        out_shape=jax.ShapeDtypeStruct((1, NG), jnp.float32),
    )(nid3)

    out = pl.pallas_call(
        _dense_body,
        grid=(NB,),
        in_specs=[
            pl.BlockSpec((BN, D), lambda i: (i, 0)),
            pl.BlockSpec((NC, BN, D), lambda i: (0, i, 0)),
            pl.BlockSpec((1, 1, BN), lambda i: (i, 0, 0)),
            pl.BlockSpec((1, NG), lambda i: (0, 0)),
            pl.BlockSpec((D, 2 * D), lambda i: (0, 0)),
            pl.BlockSpec((1, 2 * D), lambda i: (0, 0)),
            pl.BlockSpec((2 * D, D), lambda i: (0, 0)),
            pl.BlockSpec((1, D), lambda i: (0, 0)),
            pl.BlockSpec((1, D), lambda i: (0, 0)),
            pl.BlockSpec((1, D), lambda i: (0, 0)),
        ],
        out_specs=pl.BlockSpec((BN, D), lambda i: (i, 0)),
        out_shape=jax.ShapeDtypeStruct((N, D), jnp.float32),
    )(node_hidden, partials, nid3, counts,
      W1, jnp.reshape(b1, (1, 2 * D)), W2, jnp.reshape(b2, (1, D)),
      jnp.reshape(ln_gamma, (1, D)), jnp.reshape(ln_beta, (1, D)))
    return out


# speedup vs baseline: 9.0272x; 1.0935x over previous
import functools

import jax
import jax.numpy as jnp
from jax import lax
from jax.experimental import pallas as pl
from jax.experimental.pallas import tpu as pltpu
from jax.experimental.pallas import tpu_sc as plsc

N = 10000
E = 320000
D = 128
NG = 512

NC = 2
NS = 16
NW = NC * NS

K = 80
CPW = E // K // NW

NZ = N // K

BN = 1000
NB = N // BN


def _sc_aggregate_body(nh_hbm, es_hbm, ed_hbm, eh_hbm, out_hbm,
                       sx0, sx1, sx2, sx3, dx0, dx1, dx2, dx3,
                       rows0, rows1, er0, er1,
                       aggr_sh, si0, si1, si2, si3, sg0, sg1, ss0, ss1):
    cid = lax.axis_index("c")
    sid = lax.axis_index("s")
    wid = sid * NC + cid

    sidx = (sx0, sx1, sx2, sx3)
    didx = (dx0, dx1, dx2, dx3)
    rows = (rows0, rows1)
    er = (er0, er1)
    semi = (si0, si1, si2, si3)
    semg = (sg0, sg1)
    sems = (ss0, ss1)

    def base_of(c):
        return (wid * CPW + c) * K

    def issue_idx(c, s):
        pltpu.async_copy(es_hbm.at[pl.ds(base_of(c), K)], sidx[s], semi[s])
        pltpu.async_copy(ed_hbm.at[pl.ds(base_of(c), K)], didx[s], semi[s])

    def wait_idx(c, s):
        pltpu.make_async_copy(es_hbm.at[pl.ds(base_of(c), K)], sidx[s],
                              semi[s]).wait()
        pltpu.make_async_copy(ed_hbm.at[pl.ds(base_of(c), K)], didx[s],
                              semi[s]).wait()

    def issue_ge(c, b, s):
        pltpu.async_copy(nh_hbm.at[sidx[s]], rows[b], semg[b])
        pltpu.async_copy(eh_hbm.at[pl.ds(base_of(c), K)], er[b], semg[b])

    def wait_ge(c, b, s):
        pltpu.make_async_copy(nh_hbm.at[sidx[s]], rows[b], semg[b]).wait()
        pltpu.make_async_copy(eh_hbm.at[pl.ds(base_of(c), K)], er[b],
                              semg[b]).wait()

    def issue_sc(b, s):
        pltpu.async_copy(er[b], aggr_sh.at[didx[s]], sems[b], add=True)

    def wait_sc(b, s):
        pltpu.make_async_copy(er[b], aggr_sh.at[didx[s]], sems[b]).wait()

    def compute(b):
        @plsc.parallel_loop(0, K, unroll=2)
        def _(i):
            for g in range(D // 32):
                w = rows[b][i, pl.ds(g * 16, 16)]
                xa = lax.bitcast_convert_type(lax.shift_left(w, 16),
                                              jnp.float32)
                xb = lax.bitcast_convert_type(
                    jnp.bitwise_and(w, jnp.int32(-65536)), jnp.float32)
                sla = pl.ds(g * 32, 16)
                slb = pl.ds(g * 32 + 16, 16)
                er[b][i, sla] = jnp.maximum(xa + er[b][i, sla], 0.0)
                er[b][i, slb] = jnp.maximum(xb + er[b][i, slb], 0.0)

    def _zrow(i, carry):
        for j in range(D // 16):
            er0[i, pl.ds(j * 16, 16)] = jnp.zeros((16,), jnp.float32)
        return carry
    lax.fori_loop(0, K, _zrow, 0)
    for m in range(NZ):
        @pl.when(sid == (m % NS))
        def _():
            pltpu.sync_copy(er0, aggr_sh.at[pl.ds(m * K, K)])
    plsc.subcore_barrier()

    issue_idx(0, 0)
    issue_idx(1, 1)
    wait_idx(0, 0)
    issue_ge(0, 0, 0)

    def group(c2, carry):
        for bb in range(4):
            c = c2 * 4 + bb
            b = bb % 2
            ob = 1 - b

            @pl.when(c >= 1)
            def _():
                wait_sc(ob, (bb + 3) % 4)

            @pl.when(c + 2 < CPW)
            def _():
                issue_idx(c + 2, (bb + 2) % 4)

            wait_idx(c + 1, (bb + 1) % 4)
            issue_ge(c + 1, ob, (bb + 1) % 4)
            wait_ge(c, b, bb)
            compute(b)
            issue_sc(b, bb)
        return carry

    lax.fori_loop(0, (CPW - 1) // 4, group, 0)

    wait_sc(1, 3)
    wait_ge(CPW - 1, 0, 0)
    compute(0)
    issue_sc(0, 0)
    wait_sc(0, 0)

    plsc.subcore_barrier()

    for m in range(NZ):
        @pl.when(sid == (m % NS))
        def _():
            pltpu.sync_copy(aggr_sh.at[pl.ds(m * K, K)],
                            out_hbm.at[cid, pl.ds(m * K, K)])


_sc_aggregate = functools.partial(
    pl.kernel,
    out_type=jax.ShapeDtypeStruct((NC, N, D), jnp.float32),
    mesh=plsc.VectorSubcoreMesh(core_axis_name="c", subcore_axis_name="s"),
    compiler_params=pltpu.CompilerParams(use_tc_tiling_on_sc=False),
    scratch_types=[
        pltpu.VMEM((K,), jnp.int32),
        pltpu.VMEM((K,), jnp.int32),
        pltpu.VMEM((K,), jnp.int32),
        pltpu.VMEM((K,), jnp.int32),
        pltpu.VMEM((K,), jnp.int32),
        pltpu.VMEM((K,), jnp.int32),
        pltpu.VMEM((K,), jnp.int32),
        pltpu.VMEM((K,), jnp.int32),
        pltpu.VMEM((K, D // 2), jnp.int32),
        pltpu.VMEM((K, D // 2), jnp.int32),
        pltpu.VMEM((K, D), jnp.float32),
        pltpu.VMEM((K, D), jnp.float32),
        pltpu.VMEM_SHARED((N, D), jnp.float32),
        pltpu.SemaphoreType.DMA,
        pltpu.SemaphoreType.DMA,
        pltpu.SemaphoreType.DMA,
        pltpu.SemaphoreType.DMA,
        pltpu.SemaphoreType.DMA,
        pltpu.SemaphoreType.DMA,
        pltpu.SemaphoreType.DMA,
        pltpu.SemaphoreType.DMA,
    ],
)(_sc_aggregate_body)


def _counts_body(nid_ref, out_ref):
    i = pl.program_id(0)
    nid = nid_ref[0, 0, :]
    oh = (nid[:, None] == lax.broadcasted_iota(jnp.int32, (BN, NG), 1))
    colsum = jnp.sum(oh.astype(jnp.float32), axis=0)

    @pl.when(i == 0)
    def _():
        out_ref[...] = colsum[None, :]

    @pl.when(i > 0)
    def _():
        out_ref[...] = out_ref[...] + colsum[None, :]


def _dense_body(x_ref, p_ref, nid_ref, cnt_ref, w1_ref, b1_ref, w2_ref,
                b2_ref, g_ref, be_ref, out_ref):
    x = x_ref[...]
    h = x + p_ref[0] + p_ref[1]
    h = jnp.dot(h, w1_ref[...], preferred_element_type=jnp.float32) + b1_ref[...]
    h = jnp.maximum(h, 0.0)
    h = jnp.dot(h, w2_ref[...], preferred_element_type=jnp.float32) + b2_ref[...]
    mu = jnp.mean(h, axis=1, keepdims=True)
    xc = h - mu
    var = jnp.mean(xc * xc, axis=1, keepdims=True)
    h = xc * lax.rsqrt(var + 1e-5) * g_ref[...] + be_ref[...]
    nid = nid_ref[0, 0, :]
    rc = lax.rsqrt(jnp.maximum(cnt_ref[...], 1.0))
    oh = (nid[:, None] == lax.broadcasted_iota(jnp.int32, (BN, NG), 1))
    rinv = jnp.sum(oh.astype(jnp.float32) * rc, axis=1, keepdims=True)
    h = jnp.maximum(h * rinv, 0.0)
    out_ref[...] = h + x


def kernel(node_hidden, edge_index, edge_hidden, node_id, edge_id,
           W1, b1, W2, b2, ln_gamma, ln_beta):
    nh_bf = jnp.transpose(
        node_hidden.astype(jnp.bfloat16).reshape(N, D // 32, 2, 16),
        (0, 1, 3, 2))
    nh_words = lax.bitcast_convert_type(nh_bf, jnp.int32).reshape(N, D // 2)
    partials = _sc_aggregate(nh_words, edge_index[0], edge_index[1],
                             edge_hidden)

    nid3 = jnp.reshape(node_id.astype(jnp.int32), (NB, 1, BN))
    counts = pl.pallas_call(
        _counts_body,
        grid=(NB,),
        in_specs=[pl.BlockSpec((1, 1, BN), lambda i: (i, 0, 0))],
        out_specs=pl.BlockSpec((1, NG), lambda i: (0, 0)),
        out_shape=jax.ShapeDtypeStruct((1, NG), jnp.float32),
    )(nid3)

    out = pl.pallas_call(
        _dense_body,
        grid=(NB,),
        in_specs=[
            pl.BlockSpec((BN, D), lambda i: (i, 0)),
            pl.BlockSpec((NC, BN, D), lambda i: (0, i, 0)),
            pl.BlockSpec((1, 1, BN), lambda i: (i, 0, 0)),
            pl.BlockSpec((1, NG), lambda i: (0, 0)),
            pl.BlockSpec((D, 2 * D), lambda i: (0, 0)),
            pl.BlockSpec((1, 2 * D), lambda i: (0, 0)),
            pl.BlockSpec((2 * D, D), lambda i: (0, 0)),
            pl.BlockSpec((1, D), lambda i: (0, 0)),
            pl.BlockSpec((1, D), lambda i: (0, 0)),
            pl.BlockSpec((1, D), lambda i: (0, 0)),
        ],
        out_specs=pl.BlockSpec((BN, D), lambda i: (i, 0)),
        out_shape=jax.ShapeDtypeStruct((N, D), jnp.float32),
    )(node_hidden, partials, nid3, counts,
      W1, jnp.reshape(b1, (1, 2 * D)), W2, jnp.reshape(b2, (1, D)),
      jnp.reshape(ln_gamma, (1, D)), jnp.reshape(ln_beta, (1, D)))
    return out
